# baseline (speedup 1.0000x reference)
import functools

import jax
import jax.numpy as jnp
from jax.experimental import pallas as pl
from jax.experimental.pallas import tpu as pltpu

_EPS = 1e-5


def _body(x_ref, w_ref, bias_ref, o_ref, *, use_ln):
    x = x_ref[:]
    if use_ln:
        m = jnp.mean(x, axis=-1, keepdims=True)
        c = x - m
        v = jnp.mean(c * c, axis=-1, keepdims=True)
        p = jnp.dot(c.astype(jnp.bfloat16), w_ref[:],
                    preferred_element_type=jnp.float32)
        o_ref[:] = p * jax.lax.rsqrt(v + _EPS) + bias_ref[:]
    else:
        o_ref[:] = jnp.dot(x.astype(jnp.bfloat16), w_ref[:],
                           preferred_element_type=jnp.float32) + bias_ref[:]


def _ln_proj(x, w_bf16, bias2, *, use_ln, blk):
    rows, d = x.shape
    h = w_bf16.shape[1]
    body = functools.partial(_body, use_ln=use_ln)
    return pl.pallas_call(
        body,
        grid=(pl.cdiv(rows, blk),),
        in_specs=[
            pl.BlockSpec((blk, d), lambda i: (i, 0)),
            pl.BlockSpec((d, h), lambda i: (0, 0)),
            pl.BlockSpec((1, h), lambda i: (0, 0)),
        ],
        out_specs=pl.BlockSpec((blk, h), lambda i: (i, 0)),
        out_shape=jax.ShapeDtypeStruct((rows, h), jnp.float32),
        compiler_params=pltpu.CompilerParams(
            dimension_semantics=("arbitrary",)),
    )(x, w_bf16, bias2.reshape(1, h))


def _node_q_body(x_ref, q_ref, w_ref, bias_ref, qw_ref, qb_ref,
                 o_ref, qo_ref, *, last_step):
    _body(x_ref, w_ref, bias_ref, o_ref, use_ln=True)

    @pl.when(pl.program_id(0) == last_step)
    def _():
        qo_ref[:] = jnp.dot(q_ref[:].astype(jnp.bfloat16), qw_ref[:],
                            preferred_element_type=jnp.float32) + qb_ref[:]


def _node_q_proj(x, q, w_bf16, bias2, qw_bf16, qb, *, blk):
    rows, d = x.shape
    h = w_bf16.shape[1]
    b_rows = q.shape[0]
    grid = pl.cdiv(rows, blk)
    const = lambda i: (0, 0)
    body = functools.partial(_node_q_body, last_step=grid - 1)
    return pl.pallas_call(
        body,
        grid=(grid,),
        in_specs=[
            pl.BlockSpec((blk, d), lambda i: (i, 0)),
            pl.BlockSpec((b_rows, d), const),
            pl.BlockSpec((d, h), const),
            pl.BlockSpec((1, h), const),
            pl.BlockSpec((d, h), const),
            pl.BlockSpec((1, h), const),
        ],
        out_specs=[
            pl.BlockSpec((blk, h), lambda i: (i, 0)),
            pl.BlockSpec((b_rows, h), const),
        ],
        out_shape=[
            jax.ShapeDtypeStruct((rows, h), jnp.float32),
            jax.ShapeDtypeStruct((b_rows, h), jnp.float32),
        ],
        compiler_params=pltpu.CompilerParams(
            dimension_semantics=("arbitrary",)),
    )(x, q, w_bf16, bias2.reshape(1, h), qw_bf16, qb.reshape(1, h))


def kernel(node_embeddings, edge_embeddings, question_emb,
           node_norm_g, node_norm_b, rel_norm_g, rel_norm_b,
           node_W, node_b, rel_W, rel_b, q_W, q_b):
    node_Wg = (node_norm_g[:, None] * node_W).astype(jnp.bfloat16)
    node_bias2 = node_norm_b @ node_W + node_b
    rel_Wg = (rel_norm_g[:, None] * rel_W).astype(jnp.bfloat16)
    rel_bias2 = rel_norm_b @ rel_W + rel_b

    node_tokens, question_tokens = _node_q_proj(
        node_embeddings, question_emb, node_Wg, node_bias2,
        q_W.astype(jnp.bfloat16), q_b, blk=5000)
    relation_tokens = _ln_proj(edge_embeddings, rel_Wg, rel_bias2,
                               use_ln=True, blk=20000)
    return (node_tokens, relation_tokens, question_tokens)


# speedup vs baseline: 1.0095x; 1.0095x over previous
import functools

import jax
import jax.numpy as jnp
from jax.experimental import pallas as pl
from jax.experimental.pallas import tpu as pltpu

_EPS = 1e-5


def _body(x_ref, w_ref, bias_ref, o_ref, *, use_ln):
    x = x_ref[:]
    if use_ln:
        m = jnp.mean(x, axis=-1, keepdims=True)
        c = x - m
        v = jnp.mean(c * c, axis=-1, keepdims=True)
        p = jnp.dot(c.astype(jnp.bfloat16), w_ref[:],
                    preferred_element_type=jnp.float32)
        o_ref[:] = p * jax.lax.rsqrt(v + _EPS) + bias_ref[:]
    else:
        o_ref[:] = jnp.dot(x.astype(jnp.bfloat16), w_ref[:],
                           preferred_element_type=jnp.float32) + bias_ref[:]


def _ln_proj(x, w_bf16, bias2, *, use_ln, blk):
    rows, d = x.shape
    h = w_bf16.shape[1]
    body = functools.partial(_body, use_ln=use_ln)
    return pl.pallas_call(
        body,
        grid=(pl.cdiv(rows, blk),),
        in_specs=[
            pl.BlockSpec((blk, d), lambda i: (i, 0)),
            pl.BlockSpec((d, h), lambda i: (0, 0)),
            pl.BlockSpec((1, h), lambda i: (0, 0)),
        ],
        out_specs=pl.BlockSpec((blk, h), lambda i: (i, 0)),
        out_shape=jax.ShapeDtypeStruct((rows, h), jnp.float32),
        compiler_params=pltpu.CompilerParams(
            dimension_semantics=("arbitrary",)),
    )(x, w_bf16, bias2.reshape(1, h))


def kernel(node_embeddings, edge_embeddings, question_emb,
           node_norm_g, node_norm_b, rel_norm_g, rel_norm_b,
           node_W, node_b, rel_W, rel_b, q_W, q_b):
    node_Wg = (node_norm_g[:, None] * node_W).astype(jnp.bfloat16)
    node_bias2 = node_norm_b @ node_W + node_b
    rel_Wg = (rel_norm_g[:, None] * rel_W).astype(jnp.bfloat16)
    rel_bias2 = rel_norm_b @ rel_W + rel_b

    node_tokens = _ln_proj(node_embeddings, node_Wg, node_bias2,
                           use_ln=True, blk=5000)
    relation_tokens = _ln_proj(edge_embeddings, rel_Wg, rel_bias2,
                               use_ln=True, blk=20000)
    question_tokens = _ln_proj(question_emb, q_W.astype(jnp.bfloat16), q_b,
                               use_ln=False, blk=1024)
    return (node_tokens, relation_tokens, question_tokens)
